# baseline (speedup 1.0000x reference)
import functools

import jax
import jax.numpy as jnp
from jax import lax
from jax.experimental import pallas as pl
from jax.experimental.pallas import tpu as pltpu
from jax.experimental.pallas import tpu_sc as plsc

_HIDDEN = 64
_NC = 2
_NS = 16
_NW = _NC * _NS
_CHUNK = 128
_NBUF = 2


def _sc_gather(idx_grp, pe):
    nw, n_chunks, chunk = idx_grp.shape
    mesh = plsc.VectorSubcoreMesh(core_axis_name="c", subcore_axis_name="s")

    @functools.partial(
        pl.kernel,
        out_type=jax.ShapeDtypeStruct((nw, n_chunks, chunk, _HIDDEN),
                                      jnp.float32),
        mesh=mesh,
        compiler_params=pltpu.CompilerParams(use_tc_tiling_on_sc=False),
        scratch_types=[
            pltpu.VMEM((n_chunks, chunk), jnp.int32),
            pltpu.VMEM((_NBUF, chunk, _HIDDEN), jnp.float32),
            pltpu.SemaphoreType.DMA((_NBUF,)),
            pltpu.SemaphoreType.DMA((_NBUF,)),
        ],
    )
    def k(idx_hbm, pe_hbm, out_hbm, idx_v, rows_v, gsem, wsem):
        wid = lax.axis_index("s") * _NC + lax.axis_index("c")
        pltpu.sync_copy(idx_hbm.at[wid], idx_v)

        def gather_start(c, buf):
            return pltpu.async_copy(
                pe_hbm.at[idx_v.at[c]], rows_v.at[buf], gsem.at[buf])

        def write_start(c, buf):
            return pltpu.async_copy(
                rows_v.at[buf], out_hbm.at[wid, c], wsem.at[buf])

        gather_start(0, 0)

        def body(c, _):
            buf = lax.rem(c, _NBUF)
            nxt = lax.rem(c + 1, _NBUF)

            @pl.when(c + 1 < n_chunks)
            def _():
                @pl.when(c + 1 >= _NBUF)
                def _():
                    pltpu.make_async_copy(
                        rows_v.at[nxt], out_hbm.at[wid, c], wsem.at[nxt]
                    ).wait()
                gather_start(c + 1, nxt)

            pltpu.make_async_copy(
                pe_hbm.at[idx_v.at[c]], rows_v.at[buf], gsem.at[buf]
            ).wait()
            write_start(c, buf)
            return 0

        lax.fori_loop(0, n_chunks, body, 0)
        def drain(c, _):
            buf = lax.rem(c, _NBUF)
            pltpu.make_async_copy(
                rows_v.at[buf], out_hbm.at[wid, c], wsem.at[buf]
            ).wait()
            return 0
        lax.fori_loop(n_chunks - _NBUF, n_chunks, drain, 0)

    return k(idx_grp, pe)


def kernel(idx, pe):
    b, s = idx.shape
    total = b * s
    n_chunks = total // (_NW * _CHUNK)
    idx_grp = idx.reshape(_NW, n_chunks, _CHUNK)
    out = _sc_gather(idx_grp, pe)
    return out.reshape(b, s, _HIDDEN)
